# baseline (speedup 1.0000x reference)
import functools

import jax
import jax.numpy as jnp
from jax import lax
from jax.experimental import pallas as pl
from jax.experimental.pallas import tpu as pltpu
from jax.experimental.pallas import tpu_sc as plsc

N = 10000
D = 128
E = 320000

NC = 2
NS = 16
NW = NC * NS
BLK = 128
BPT0 = 160
BPT1 = 0
NBLK = NS * (BPT0 + BPT1)
E_PAD = NBLK * BLK
N_PAD = 10240
ZROWS = N_PAD // NS
NG = 10
GB0 = BPT0 // NG
GB1 = BPT1 // NG
IDX_ROWS = NBLK + GB0 - GB1

@functools.cache
def _make_sc_edge_agg():
    mesh = plsc.VectorSubcoreMesh(core_axis_name="c", subcore_axis_name="s")
    return pl.kernel(
        _sc_edge_agg_body,
        mesh=mesh,
        out_type=jax.ShapeDtypeStruct((NC, N_PAD, D), jnp.float32),
        scratch_types=[
            pltpu.VMEM((2, GB0, BLK), jnp.int32),
            pltpu.VMEM((2, GB0, BLK), jnp.int32),
            pltpu.VMEM((BLK, D), jnp.float32),
            pltpu.VMEM((BLK, D), jnp.float32),
            pltpu.VMEM_SHARED((N_PAD, D), jnp.float32),
            pltpu.SemaphoreType.DMA,
            pltpu.SemaphoreType.DMA,
            pltpu.SemaphoreType.DMA,
            pltpu.SemaphoreType.DMA,
            pltpu.SemaphoreType.DMA,
        ],
    )


def _sc_edge_agg_body(y_hbm, src_hbm, dst_hbm, zeros_hbm, out_hbm,
                      src_v, dst_v, rows0, rows1, acc_sh,
                      gsem0, gsem1, ssem0, ssem1, isem):
    c = lax.axis_index("c")
    s = lax.axis_index("s")
    gb = GB0 - c * (GB0 - GB1)
    base_blk = c * (NS * BPT0) + s * (gb * NG)

    def idx_load(g, buf):
        base = base_blk + g * gb
        pltpu.async_copy(src_hbm.at[pl.ds(base, GB0)], src_v.at[buf], isem)
        pltpu.async_copy(dst_hbm.at[pl.ds(base, GB0)], dst_v.at[buf], isem)

    def idx_wait(g, buf):
        base = base_blk + g * gb
        pltpu.make_async_copy(src_hbm.at[pl.ds(base, GB0)], src_v.at[buf], isem).wait()
        pltpu.make_async_copy(dst_hbm.at[pl.ds(base, GB0)], dst_v.at[buf], isem).wait()

    def fire_gather(buf, jj, rows, gsem):
        pltpu.async_copy(y_hbm.at[src_v.at[buf, jj]], rows, gsem)

    def wait_gather(buf, jj, rows, gsem):
        pltpu.make_async_copy(y_hbm.at[src_v.at[buf, jj]], rows, gsem).wait()

    def fire_scatter(buf, jj, rows, ssem):
        pltpu.async_copy(rows, acc_sh.at[dst_v.at[buf, jj]], ssem, add=True)

    def wait_scatter(buf, jj, rows, ssem):
        pltpu.make_async_copy(rows, acc_sh.at[dst_v.at[buf, jj]], ssem).wait()

    pltpu.sync_copy(zeros_hbm.at[pl.ds(s * ZROWS, ZROWS)],
                    acc_sh.at[pl.ds(s * ZROWS, ZROWS)])
    plsc.subcore_barrier()

    @pl.when(gb > 0)
    def _stream():
        idx_load(0, 0)
        idx_wait(0, 0)
        for g in range(NG):
            buf = g % 2
            if g + 1 < NG:
                idx_load(g + 1, 1 - buf)

            fire_gather(buf, 0, rows0, gsem0)
            fire_gather(buf, 1, rows1, gsem1)
            wait_gather(buf, 0, rows0, gsem0)
            fire_scatter(buf, 0, rows0, ssem0)

            def body(i, carry, buf=buf):
                j1 = 1 + 2 * i
                wait_scatter(buf, j1 - 1, rows0, ssem0)
                fire_gather(buf, j1 + 1, rows0, gsem0)
                wait_gather(buf, j1, rows1, gsem1)
                fire_scatter(buf, j1, rows1, ssem1)
                j2 = j1 + 1
                wait_scatter(buf, j2 - 1, rows1, ssem1)
                fire_gather(buf, j2 + 1, rows1, gsem1)
                wait_gather(buf, j2, rows0, gsem0)
                fire_scatter(buf, j2, rows0, ssem0)
                return carry

            lax.fori_loop(0, (gb - 2) // 2, body, 0)

            wait_scatter(buf, gb - 2, rows0, ssem0)
            wait_gather(buf, gb - 1, rows1, gsem1)
            fire_scatter(buf, gb - 1, rows1, ssem1)
            wait_scatter(buf, gb - 1, rows1, ssem1)

            if g + 1 < NG:
                idx_wait(g + 1, 1 - buf)

    plsc.subcore_barrier()
    pltpu.sync_copy(acc_sh.at[pl.ds(s * ZROWS, ZROWS)],
                    out_hbm.at[c, pl.ds(s * ZROWS, ZROWS)])


def _dense_in_body(x_ref, wr_ref, wo_ref, b_ref, y_ref, r_ref):
    xb = x_ref[...]
    y_ref[...] = jnp.dot(xb, wr_ref[...], preferred_element_type=jnp.float32)
    r_ref[...] = (jnp.dot(xb, wo_ref[...], preferred_element_type=jnp.float32)
                  + b_ref[...])


def _dense_mid_body(p_ref, r_ref, wr_ref, wo_ref, b_ref, y_ref, r2_ref):
    h = jnp.maximum(p_ref[0] + p_ref[1] + r_ref[...], 0.0)
    y_ref[...] = jnp.dot(h, wr_ref[...], preferred_element_type=jnp.float32)
    r2_ref[...] = (jnp.dot(h, wo_ref[...], preferred_element_type=jnp.float32)
                   + b_ref[...])


def _final_body(q_ref, r_ref, o_ref):
    o_ref[...] = q_ref[0] + q_ref[1] + r_ref[...]


_ROWB = 2000
_GRID = N // _ROWB

_w_spec = pl.BlockSpec((D, D), lambda i: (0, 0))
_b_spec = pl.BlockSpec((1, D), lambda i: (0, 0))
_x_spec = pl.BlockSpec((_ROWB, D), lambda i: (i, 0))
_p_spec = pl.BlockSpec((NC, _ROWB, D), lambda i: (0, i, 0))
_nd = jax.ShapeDtypeStruct((N, D), jnp.float32)

_dense_in = pl.pallas_call(
    _dense_in_body,
    grid=(_GRID,),
    in_specs=[_x_spec, _w_spec, _w_spec, _b_spec],
    out_specs=[_x_spec, _x_spec],
    out_shape=[_nd, _nd],
)

_dense_mid = pl.pallas_call(
    _dense_mid_body,
    grid=(_GRID,),
    in_specs=[_p_spec, _x_spec, _w_spec, _w_spec, _b_spec],
    out_specs=[_x_spec, _x_spec],
    out_shape=[_nd, _nd],
)

_final = pl.pallas_call(
    _final_body,
    grid=(_GRID,),
    in_specs=[_p_spec, _x_spec],
    out_specs=_x_spec,
    out_shape=_nd,
)


def kernel(x, edge_index, W1_rel, b1, W1_root, W2_rel, b2, W2_root):
    src = edge_index[0].astype(jnp.int32)
    dst = edge_index[1].astype(jnp.int32)
    pad = E_PAD - E
    src_p = jnp.concatenate([src, jnp.zeros((pad,), jnp.int32)])
    dst_p = jnp.concatenate([dst, jnp.full((pad,), N, jnp.int32)])
    rpad = jnp.zeros((IDX_ROWS - NBLK, BLK), jnp.int32)
    src_p = jnp.concatenate([src_p.reshape(NBLK, BLK), rpad])
    dst_p = jnp.concatenate([dst_p.reshape(NBLK, BLK), rpad])
    zeros = jnp.zeros((N_PAD, D), jnp.float32)
    b1r = b1.reshape(1, D)
    b2r = b2.reshape(1, D)

    sc_edge_agg = _make_sc_edge_agg()
    y1, r1 = _dense_in(x, W1_rel, W1_root, b1r)
    p = sc_edge_agg(y1, src_p, dst_p, zeros)
    y2, r2 = _dense_mid(p, r1, W2_rel, W2_root, b2r)
    q = sc_edge_agg(y2, src_p, dst_p, zeros)
    return _final(q, r2)


# speedup vs baseline: 1.2430x; 1.2430x over previous
import functools

import jax
import jax.numpy as jnp
from jax import lax
from jax.experimental import pallas as pl
from jax.experimental.pallas import tpu as pltpu
from jax.experimental.pallas import tpu_sc as plsc

N = 10000
D = 128
E = 320000

NC = 2
NS = 16
NW = NC * NS
BLK = 128
BPT0 = 128
BPT1 = 32
NBLK = NS * (BPT0 + BPT1)
E_PAD = NBLK * BLK
N_PAD = 10240
ZROWS = N_PAD // NS
NG = 4
GB0 = BPT0 // NG
GB1 = BPT1 // NG
IDX_ROWS = NBLK + GB0 - GB1

@functools.cache
def _make_sc_edge_agg():
    mesh = plsc.VectorSubcoreMesh(core_axis_name="c", subcore_axis_name="s")
    return pl.kernel(
        _sc_edge_agg_body,
        mesh=mesh,
        out_type=jax.ShapeDtypeStruct((NC, N_PAD, D), jnp.float32),
        scratch_types=[
            pltpu.VMEM((2, GB0, BLK), jnp.int32),
            pltpu.VMEM((2, GB0, BLK), jnp.int32),
            pltpu.VMEM((BLK, D), jnp.float32),
            pltpu.VMEM((BLK, D), jnp.float32),
            pltpu.VMEM_SHARED((N_PAD, D), jnp.float32),
            pltpu.SemaphoreType.DMA,
            pltpu.SemaphoreType.DMA,
            pltpu.SemaphoreType.DMA,
            pltpu.SemaphoreType.DMA,
            pltpu.SemaphoreType.DMA,
        ],
    )


def _sc_edge_agg_body(y_hbm, src_hbm, dst_hbm, zeros_hbm, out_hbm,
                      src_v, dst_v, rows0, rows1, acc_sh,
                      gsem0, gsem1, ssem0, ssem1, isem):
    c = lax.axis_index("c")
    s = lax.axis_index("s")
    gb = GB0 - c * (GB0 - GB1)
    base_blk = c * (NS * BPT0) + s * (gb * NG)

    def idx_load(g, buf):
        base = base_blk + g * gb
        pltpu.async_copy(src_hbm.at[pl.ds(base, GB0)], src_v.at[buf], isem)
        pltpu.async_copy(dst_hbm.at[pl.ds(base, GB0)], dst_v.at[buf], isem)

    def idx_wait(g, buf):
        base = base_blk + g * gb
        pltpu.make_async_copy(src_hbm.at[pl.ds(base, GB0)], src_v.at[buf], isem).wait()
        pltpu.make_async_copy(dst_hbm.at[pl.ds(base, GB0)], dst_v.at[buf], isem).wait()

    def fire_gather(buf, jj, rows, gsem):
        pltpu.async_copy(y_hbm.at[src_v.at[buf, jj]], rows, gsem)

    def wait_gather(buf, jj, rows, gsem):
        pltpu.make_async_copy(y_hbm.at[src_v.at[buf, jj]], rows, gsem).wait()

    def fire_scatter(buf, jj, rows, ssem):
        pltpu.async_copy(rows, acc_sh.at[dst_v.at[buf, jj]], ssem, add=True)

    def wait_scatter(buf, jj, rows, ssem):
        pltpu.make_async_copy(rows, acc_sh.at[dst_v.at[buf, jj]], ssem).wait()

    pltpu.sync_copy(zeros_hbm.at[pl.ds(s * ZROWS, ZROWS)],
                    acc_sh.at[pl.ds(s * ZROWS, ZROWS)])
    plsc.subcore_barrier()

    @pl.when(gb > 0)
    def _stream():
        idx_load(0, 0)
        idx_wait(0, 0)
        for g in range(NG):
            buf = g % 2
            if g + 1 < NG:
                idx_load(g + 1, 1 - buf)

            fire_gather(buf, 0, rows0, gsem0)
            fire_gather(buf, 1, rows1, gsem1)
            wait_gather(buf, 0, rows0, gsem0)
            fire_scatter(buf, 0, rows0, ssem0)

            def body(i, carry, buf=buf):
                j1 = 1 + 2 * i
                wait_scatter(buf, j1 - 1, rows0, ssem0)
                fire_gather(buf, j1 + 1, rows0, gsem0)
                wait_gather(buf, j1, rows1, gsem1)
                fire_scatter(buf, j1, rows1, ssem1)
                j2 = j1 + 1
                wait_scatter(buf, j2 - 1, rows1, ssem1)
                fire_gather(buf, j2 + 1, rows1, gsem1)
                wait_gather(buf, j2, rows0, gsem0)
                fire_scatter(buf, j2, rows0, ssem0)
                return carry

            lax.fori_loop(0, (gb - 2) // 2, body, 0)

            wait_scatter(buf, gb - 2, rows0, ssem0)
            wait_gather(buf, gb - 1, rows1, gsem1)
            fire_scatter(buf, gb - 1, rows1, ssem1)
            wait_scatter(buf, gb - 1, rows1, ssem1)

            if g + 1 < NG:
                idx_wait(g + 1, 1 - buf)

    plsc.subcore_barrier()
    pltpu.sync_copy(acc_sh.at[pl.ds(s * ZROWS, ZROWS)],
                    out_hbm.at[c, pl.ds(s * ZROWS, ZROWS)])


def _dense_in_body(x_ref, wr_ref, wo_ref, b_ref, y_ref, r_ref):
    xb = x_ref[...]
    y_ref[...] = jnp.dot(xb, wr_ref[...], preferred_element_type=jnp.float32)
    r_ref[...] = (jnp.dot(xb, wo_ref[...], preferred_element_type=jnp.float32)
                  + b_ref[...])


def _dense_mid_body(p_ref, r_ref, wr_ref, wo_ref, b_ref, y_ref, r2_ref):
    h = jnp.maximum(p_ref[0] + p_ref[1] + r_ref[...], 0.0)
    y_ref[...] = jnp.dot(h, wr_ref[...], preferred_element_type=jnp.float32)
    r2_ref[...] = (jnp.dot(h, wo_ref[...], preferred_element_type=jnp.float32)
                   + b_ref[...])


def _final_body(q_ref, r_ref, o_ref):
    o_ref[...] = q_ref[0] + q_ref[1] + r_ref[...]


_ROWB = 2000
_GRID = N // _ROWB

_w_spec = pl.BlockSpec((D, D), lambda i: (0, 0))
_b_spec = pl.BlockSpec((1, D), lambda i: (0, 0))
_x_spec = pl.BlockSpec((_ROWB, D), lambda i: (i, 0))
_p_spec = pl.BlockSpec((NC, _ROWB, D), lambda i: (0, i, 0))
_nd = jax.ShapeDtypeStruct((N, D), jnp.float32)

_dense_in = pl.pallas_call(
    _dense_in_body,
    grid=(_GRID,),
    in_specs=[_x_spec, _w_spec, _w_spec, _b_spec],
    out_specs=[_x_spec, _x_spec],
    out_shape=[_nd, _nd],
)

_dense_mid = pl.pallas_call(
    _dense_mid_body,
    grid=(_GRID,),
    in_specs=[_p_spec, _x_spec, _w_spec, _w_spec, _b_spec],
    out_specs=[_x_spec, _x_spec],
    out_shape=[_nd, _nd],
)

_final = pl.pallas_call(
    _final_body,
    grid=(_GRID,),
    in_specs=[_p_spec, _x_spec],
    out_specs=_x_spec,
    out_shape=_nd,
)


def kernel(x, edge_index, W1_rel, b1, W1_root, W2_rel, b2, W2_root):
    src = edge_index[0].astype(jnp.int32)
    dst = edge_index[1].astype(jnp.int32)
    pad = E_PAD - E
    src_p = jnp.concatenate([src, jnp.zeros((pad,), jnp.int32)])
    dst_p = jnp.concatenate([dst, jnp.full((pad,), N, jnp.int32)])
    rpad = jnp.zeros((IDX_ROWS - NBLK, BLK), jnp.int32)
    src_p = jnp.concatenate([src_p.reshape(NBLK, BLK), rpad])
    dst_p = jnp.concatenate([dst_p.reshape(NBLK, BLK), rpad])
    zeros = jnp.zeros((N_PAD, D), jnp.float32)
    b1r = b1.reshape(1, D)
    b2r = b2.reshape(1, D)

    sc_edge_agg = _make_sc_edge_agg()
    y1, r1 = _dense_in(x, W1_rel, W1_root, b1r)
    p = sc_edge_agg(y1, src_p, dst_p, zeros)
    y2, r2 = _dense_mid(p, r1, W2_rel, W2_root, b2r)
    q = sc_edge_agg(y2, src_p, dst_p, zeros)
    return _final(q, r2)
